# Initial kernel scaffold; baseline (speedup 1.0000x reference)
import jax
import jax.numpy as jnp
from jax.experimental import pallas as pl


def kernel(unknown, known, unknow_feats, known_feats, W, b):
    raise NotImplementedError("write your pallas kernel here")


# speedup vs baseline: 23.8911x; 23.8911x over previous
import functools

import jax
import jax.numpy as jnp
from jax.experimental import pallas as pl

B, N, M, C1, C2, CO = 4, 8192, 1024, 32, 64, 128
TN = 512


def _fp_kernel(u_ref, k_ref, uf_ref, kf_ref, w_ref, b_ref, o_ref):
    u = u_ref[0]
    kpts = k_ref[0]

    e0 = (u[:, 0:1] - kpts[:, 0][None, :]) ** 2
    e1 = (u[:, 1:2] - kpts[:, 1][None, :]) ** 2
    e2 = (u[:, 2:3] - kpts[:, 2][None, :]) ** 2
    d2 = (e0 + e1) + e2

    iota = jax.lax.broadcasted_iota(jnp.int32, (TN, M), 1)

    vals = []
    idxs = []
    for _ in range(3):
        mval = jnp.min(d2, axis=1, keepdims=True)
        hit = d2 == mval
        ji = jnp.min(jnp.where(hit, iota, M), axis=1)
        vals.append(mval[:, 0])
        idxs.append(ji)
        d2 = jnp.where(iota == ji[:, None], jnp.inf, d2)

    rs = [1.0 / (jnp.sqrt(jnp.maximum(v, 0.0)) + 1e-8) for v in vals]
    norm = (rs[0] + rs[1]) + rs[2]
    ws = [r / norm for r in rs]

    iota_m = jax.lax.broadcasted_iota(jnp.int32, (M, TN), 0)
    s = jnp.where(iota_m == idxs[0][None, :], ws[0][None, :], 0.0)
    s = s + jnp.where(iota_m == idxs[1][None, :], ws[1][None, :], 0.0)
    s = s + jnp.where(iota_m == idxs[2][None, :], ws[2][None, :], 0.0)

    kf = kf_ref[0]
    interp = jnp.dot(kf, s, preferred_element_type=jnp.float32)

    x = jnp.concatenate([interp, uf_ref[0]], axis=0)
    out = jnp.dot(w_ref[...], x, preferred_element_type=jnp.float32)
    out = out + b_ref[...]
    o_ref[0] = jnp.maximum(out, 0.0)


@jax.jit
def kernel(unknown, known, unknow_feats, known_feats, W, b):
    grid = (B, N // TN)
    out = pl.pallas_call(
        _fp_kernel,
        grid=grid,
        in_specs=[
            pl.BlockSpec((1, TN, 3), lambda bb, i: (bb, i, 0)),
            pl.BlockSpec((1, M, 3), lambda bb, i: (bb, 0, 0)),
            pl.BlockSpec((1, C1, TN), lambda bb, i: (bb, 0, i)),
            pl.BlockSpec((1, C2, M), lambda bb, i: (bb, 0, 0)),
            pl.BlockSpec((CO, C1 + C2), lambda bb, i: (0, 0)),
            pl.BlockSpec((CO, 1), lambda bb, i: (0, 0)),
        ],
        out_specs=pl.BlockSpec((1, CO, TN), lambda bb, i: (bb, 0, i)),
        out_shape=jax.ShapeDtypeStruct((B, CO, N), jnp.float32),
    )(unknown, known, unknow_feats, known_feats, W, b.reshape(CO, 1))
    return out
